# baseline (speedup 1.0000x reference)
import functools

import jax
import jax.numpy as jnp
from jax import lax
from jax.experimental import pallas as pl
from jax.experimental.pallas import tpu as pltpu
from jax.experimental.pallas import tpu_sc as plsc

_NUM_CORES = 2
_NUM_SUBCORES = 16
_NUM_WORKERS = _NUM_CORES * _NUM_SUBCORES
_WB = 16
_TC = 2048


def _transpose_w(wt, n, dim):
    sub = _TC // 32

    def body(i_ref, o_ref):
        o_ref[...] = i_ref[...].T.reshape(sub, 32, dim)

    grid = (n + _TC - 1) // _TC
    return pl.pallas_call(
        body,
        grid=(grid,),
        in_specs=[pl.BlockSpec((dim, _TC), lambda i: (0, i))],
        out_specs=pl.BlockSpec((sub, 32, dim), lambda i: (i, 0, 0)),
        out_shape=jax.ShapeDtypeStruct((n // 32, 32, dim), wt.dtype),
    )(wt)


def _gather(table, idx, batch, hist, dim):
    rows_per_w = batch // _NUM_WORKERS
    steps = rows_per_w // _WB
    assert steps % 2 == 0 and steps * _WB == rows_per_w
    mesh = plsc.VectorSubcoreMesh(core_axis_name="c", subcore_axis_name="s")

    @functools.partial(
        pl.kernel,
        mesh=mesh,
        out_type=jax.ShapeDtypeStruct((batch, hist, dim), table.dtype),
        scratch_types=[
            pltpu.VMEM((_WB, hist), jnp.int32),
            pltpu.VMEM((_WB, hist), jnp.int32),
            pltpu.VMEM((_WB, hist, dim), jnp.float32),
            pltpu.VMEM((_WB, hist, dim), jnp.float32),
            pltpu.SemaphoreType.DMA,
            pltpu.SemaphoreType.DMA,
            pltpu.SemaphoreType.DMA,
            pltpu.SemaphoreType.DMA,
            pltpu.SemaphoreType.DMA,
            pltpu.SemaphoreType.DMA,
        ],
        compiler_params=pltpu.CompilerParams(use_tc_tiling_on_sc=False),
    )
    def k(table_hbm, idx_hbm, out_hbm,
          i0, i1, r0, r1, si0, si1, sg0, sg1, sw0, sw1):
        wid = lax.axis_index("s") * _NUM_CORES + lax.axis_index("c")
        base = wid * rows_per_w
        bufs = ((i0, si0, r0, sg0, sw0), (i1, si1, r1, sg1, sw1))

        pltpu.async_copy(idx_hbm.at[pl.ds(base, _WB)], i0, si0)
        pltpu.async_copy(idx_hbm.at[pl.ds(base + _WB, _WB)], i1, si1)

        @pl.loop(0, steps, step=2)
        def _(t):
            for b, (iv, si, rv, sg, sw) in enumerate(bufs):
                row = base + (t + b) * _WB
                pltpu.make_async_copy(
                    idx_hbm.at[pl.ds(row, _WB)], iv, si).wait()

                @pl.when(t + b >= 2)
                def _():
                    pltpu.make_async_copy(
                        rv, out_hbm.at[pl.ds(row - 2 * _WB, _WB)], sw).wait()

                for r in range(_WB):
                    pltpu.async_copy(
                        table_hbm.at[iv.at[r]], rv.at[r], sg)

            for b, (iv, si, rv, sg, sw) in enumerate(bufs):
                row = base + (t + b) * _WB
                for r in range(_WB):
                    pltpu.make_async_copy(
                        table_hbm.at[iv.at[r]], rv.at[r], sg).wait()

                @pl.when(t + b + 2 < steps)
                def _():
                    pltpu.async_copy(
                        idx_hbm.at[pl.ds(row + 2 * _WB, _WB)], iv, si)

                pltpu.async_copy(rv, out_hbm.at[pl.ds(row, _WB)], sw)

        pltpu.make_async_copy(
            r0, out_hbm.at[pl.ds(base + (steps - 2) * _WB, _WB)], sw0).wait()
        pltpu.make_async_copy(
            r1, out_hbm.at[pl.ds(base + (steps - 1) * _WB, _WB)], sw1).wait()

    return k(table, idx)


def kernel(input_, weight):
    batch, hist = input_.shape
    n, dim = weight.shape
    table = _transpose_w(weight.T, n, dim).reshape(n, dim)
    return _gather(table, input_.astype(jnp.int32), batch, hist, dim)


# speedup vs baseline: 1.7190x; 1.7190x over previous
import functools

import jax
import jax.numpy as jnp
from jax import lax
from jax.experimental import pallas as pl
from jax.experimental.pallas import tpu as pltpu
from jax.experimental.pallas import tpu_sc as plsc

_NUM_CORES = 2
_NUM_SUBCORES = 16
_NUM_WORKERS = _NUM_CORES * _NUM_SUBCORES
_WB = 16
_TB = 512


def _transpose_out(flat, batch, hist, dim):
    hd = hist * dim

    def body(i_ref, o_ref):
        o_ref[...] = i_ref[...].T

    return pl.pallas_call(
        body,
        grid=(batch // _TB,),
        in_specs=[pl.BlockSpec((_TB, hd), lambda i: (i, 0))],
        out_specs=pl.BlockSpec((hd, _TB), lambda i: (0, i)),
        out_shape=jax.ShapeDtypeStruct((hd, batch), flat.dtype),
    )(flat)


def _gather(table, idx, batch, hist, dim):
    rows_per_w = batch // _NUM_WORKERS
    steps = rows_per_w // _WB
    assert steps % 2 == 0 and steps * _WB == rows_per_w
    n_idx = _WB * hist
    mesh = plsc.VectorSubcoreMesh(core_axis_name="c", subcore_axis_name="s")

    @functools.partial(
        pl.kernel,
        mesh=mesh,
        out_type=jax.ShapeDtypeStruct((batch * hist, dim), table.dtype),
        scratch_types=[
            pltpu.VMEM((_WB, hist), jnp.int32),
            pltpu.VMEM((_WB, hist), jnp.int32),
            pltpu.VMEM((n_idx, dim), jnp.float32),
            pltpu.VMEM((n_idx, dim), jnp.float32),
            pltpu.SemaphoreType.DMA,
            pltpu.SemaphoreType.DMA,
            pltpu.SemaphoreType.DMA,
            pltpu.SemaphoreType.DMA,
            pltpu.SemaphoreType.DMA,
            pltpu.SemaphoreType.DMA,
        ],
        compiler_params=pltpu.CompilerParams(use_tc_tiling_on_sc=False),
    )
    def k(table_hbm, idx_hbm, out_hbm,
          i0, i1, r0, r1, si0, si1, sg0, sg1, sw0, sw1):
        wid = lax.axis_index("s") * _NUM_CORES + lax.axis_index("c")
        base = wid * rows_per_w
        bufs = ((i0, si0, r0, sg0, sw0), (i1, si1, r1, sg1, sw1))

        pltpu.async_copy(idx_hbm.at[pl.ds(base, _WB)], i0, si0)
        pltpu.async_copy(idx_hbm.at[pl.ds(base + _WB, _WB)], i1, si1)

        @pl.loop(0, steps, step=2)
        def _(t):
            for b, (iv, si, rv, sg, sw) in enumerate(bufs):
                row = base + (t + b) * _WB
                pltpu.make_async_copy(
                    idx_hbm.at[pl.ds(row, _WB)], iv, si).wait()

                @pl.when(t + b >= 2)
                def _():
                    pltpu.make_async_copy(
                        rv, out_hbm.at[pl.ds((row - 2 * _WB) * hist, n_idx)],
                        sw).wait()

                for r in range(_WB):
                    pltpu.async_copy(
                        table_hbm.at[iv.at[r]],
                        rv.at[pl.ds(r * hist, hist)], sg)

            for b, (iv, si, rv, sg, sw) in enumerate(bufs):
                row = base + (t + b) * _WB
                for r in range(_WB):
                    pltpu.make_async_copy(
                        table_hbm.at[iv.at[r]],
                        rv.at[pl.ds(r * hist, hist)], sg).wait()

                @pl.when(t + b + 2 < steps)
                def _():
                    pltpu.async_copy(
                        idx_hbm.at[pl.ds(row + 2 * _WB, _WB)], iv, si)

                pltpu.async_copy(
                    rv, out_hbm.at[pl.ds(row * hist, n_idx)], sw)

        fbase = base * hist
        pltpu.make_async_copy(
            r0, out_hbm.at[pl.ds(fbase + (steps - 2) * n_idx, n_idx)],
            sw0).wait()
        pltpu.make_async_copy(
            r1, out_hbm.at[pl.ds(fbase + (steps - 1) * n_idx, n_idx)],
            sw1).wait()

    return k(table, idx)


def kernel(input_, weight):
    batch, hist = input_.shape
    dim = weight.shape[1]
    flat = _gather(weight, input_.astype(jnp.int32), batch, hist, dim)
    flat2 = flat.reshape(batch, hist * dim)
    outt = _transpose_out(flat2, batch, hist, dim)
    return outt.reshape(hist, dim, batch).transpose(2, 0, 1)
